# Initial kernel scaffold; baseline (speedup 1.0000x reference)
import jax
import jax.numpy as jnp
from jax.experimental import pallas as pl


def kernel(h, batch_idx, W, b):
    raise NotImplementedError("write your pallas kernel here")


# speedup vs baseline: 2.6240x; 2.6240x over previous
import functools

import jax
import jax.numpy as jnp
from jax import lax
from jax.experimental import pallas as pl
from jax.experimental.pallas import tpu as pltpu
from jax.experimental.pallas import tpu_sc as plsc

N = 100000
D = 128
S = 256
NC, NS = 2, 16
NW = NC * NS
CHUNK = 80
NCHUNKS = N // CHUNK
CHUNKS_PER_TILE = (NCHUNKS + NW - 1) // NW
ROWS_PER_TILE = S // NS

_MESH = plsc.VectorSubcoreMesh(core_axis_name="c", subcore_axis_name="s")


@functools.partial(
    pl.kernel,
    out_type=jax.ShapeDtypeStruct((NC, S, D), jnp.float32),
    mesh=_MESH,
    scratch_types=[
        pltpu.VMEM((CHUNK,), jnp.int32),
        pltpu.VMEM((CHUNK, D), jnp.float32),
        pltpu.VMEM((ROWS_PER_TILE, D), jnp.float32),
        pltpu.VMEM_SHARED((S, D), jnp.float32),
    ],
)
def _segment_sum(h_hbm, idx_hbm, out_hbm, idx_v, rows_v, zbuf_v, acc_sh):
    c = lax.axis_index("c")
    s = lax.axis_index("s")
    w = s * NC + c

    zero = jnp.zeros((16,), jnp.float32)

    @pl.loop(0, ROWS_PER_TILE)
    def _zero_row(r):
        for j in range(D // 16):
            zbuf_v[r, pl.ds(j * 16, 16)] = zero

    pltpu.sync_copy(zbuf_v, acc_sh.at[pl.ds(s * ROWS_PER_TILE, ROWS_PER_TILE)])
    plsc.subcore_barrier()

    @pl.loop(0, CHUNKS_PER_TILE)
    def _chunk(i):
        g = w + NW * i

        @pl.when(g < NCHUNKS)
        def _():
            base = g * CHUNK
            pltpu.sync_copy(idx_hbm.at[pl.ds(base, CHUNK)], idx_v)
            pltpu.sync_copy(h_hbm.at[pl.ds(base, CHUNK)], rows_v)
            pltpu.sync_copy(rows_v, acc_sh.at[idx_v], add=True)

    plsc.subcore_barrier()
    pltpu.sync_copy(
        acc_sh.at[pl.ds(s * ROWS_PER_TILE, ROWS_PER_TILE)],
        out_hbm.at[c, pl.ds(s * ROWS_PER_TILE, ROWS_PER_TILE)],
    )


def _linear_relu_body(p_ref, w_ref, b_ref, o_ref):
    sums = p_ref[0] + p_ref[1]
    acc = lax.dot_general(
        sums,
        w_ref[...],
        (((1,), (1,)), ((), ())),
        preferred_element_type=jnp.float32,
        precision=lax.Precision.HIGHEST,
    )
    o_ref[...] = jnp.maximum(acc + b_ref[...], 0.0)


@functools.partial(
    pl.kernel,
    out_type=jax.ShapeDtypeStruct((N, D), jnp.float32),
    mesh=_MESH,
    scratch_types=[
        pltpu.VMEM((CHUNK,), jnp.int32),
        pltpu.VMEM((CHUNK, D), jnp.float32),
        pltpu.VMEM((CHUNK, D), jnp.float32),
        pltpu.VMEM_SHARED((S, D), jnp.float32),
    ],
)
def _broadcast_add(h_hbm, idx_hbm, hg_hbm, out_hbm, idx_v, hrows_v, grows_v, hg_sh):
    c = lax.axis_index("c")
    s = lax.axis_index("s")
    w = s * NC + c

    stage = pl.ds(s * ROWS_PER_TILE, ROWS_PER_TILE)
    pltpu.sync_copy(hg_hbm.at[stage], grows_v.at[pl.ds(0, ROWS_PER_TILE)])
    pltpu.sync_copy(grows_v.at[pl.ds(0, ROWS_PER_TILE)], hg_sh.at[stage])
    plsc.subcore_barrier()

    @pl.loop(0, CHUNKS_PER_TILE)
    def _chunk(i):
        g = w + NW * i

        @pl.when(g < NCHUNKS)
        def _():
            base = g * CHUNK
            pltpu.sync_copy(idx_hbm.at[pl.ds(base, CHUNK)], idx_v)
            pltpu.sync_copy(h_hbm.at[pl.ds(base, CHUNK)], hrows_v)
            pltpu.sync_copy(hg_sh.at[idx_v], grows_v)

            @pl.loop(0, CHUNK)
            def _row(r):
                for j in range(D // 16):
                    sl = pl.ds(j * 16, 16)
                    grows_v[r, sl] = grows_v[r, sl] + hrows_v[r, sl]

            pltpu.sync_copy(grows_v, out_hbm.at[pl.ds(base, CHUNK)])


def kernel(h, batch_idx, W, b):
    idx = batch_idx.astype(jnp.int32)
    partials = _segment_sum(h, idx)
    h_G = pl.pallas_call(
        _linear_relu_body,
        out_shape=jax.ShapeDtypeStruct((S, D), jnp.float32),
    )(partials, W, b.reshape(1, D))
    return _broadcast_add(h, idx, h_G)
